# Initial kernel scaffold; baseline (speedup 1.0000x reference)
import jax
import jax.numpy as jnp
from jax.experimental import pallas as pl


def kernel(node_feats, edge_feats, edge_index, Wn, We, attn):
    raise NotImplementedError("write your pallas kernel here")


# speedup vs baseline: 27.1269x; 27.1269x over previous
import functools

import jax
import jax.numpy as jnp
from jax import lax
from jax.experimental import pallas as pl
from jax.experimental.pallas import tpu as pltpu
from jax.experimental.pallas import tpu_sc as plsc

N = 10000
E = 160000
H = 4
F = 64
DN = 256
DE = 16

NB = 10
NBLK = N // NB
EBLK = E // NB
W = 144


def _tc_proj_body(nf, ef, wn, we, attn, hc_r, ns_r, ae_r):
    hblk = jnp.dot(nf[...], wn[...], preferred_element_type=jnp.float32)
    ztail = jnp.zeros((NBLK, W - 128), jnp.float32)
    hc_r[0] = jnp.concatenate([hblk[:, :128], ztail], axis=1)
    hc_r[1] = jnp.concatenate([hblk[:, 128:], ztail], axis=1)
    att = attn[0]
    cols = []
    for h in range(H):
        hb = hblk[:, h * F:(h + 1) * F]
        cols.append(jnp.sum(hb * att[h, 0:F][None, :], axis=1, keepdims=True))
    for h in range(H):
        hb = hblk[:, h * F:(h + 1) * F]
        cols.append(jnp.sum(hb * att[h, F:2 * F][None, :], axis=1, keepdims=True))
    cols.append(jnp.zeros((NBLK, 8), jnp.float32))
    ns_r[...] = jnp.concatenate(cols, axis=1)
    wea_cols = [
        jnp.sum(we[...][:, h * F:(h + 1) * F] * att[h, 2 * F:3 * F][None, :],
                axis=1, keepdims=True)
        for h in range(H)
    ]
    we_a = jnp.concatenate(wea_cols, axis=1)
    ae_r[...] = jnp.dot(ef[...], we_a, preferred_element_type=jnp.float32)


def _tc_proj(node_feats, edge_feats, wn, we, attn):
    return pl.pallas_call(
        _tc_proj_body,
        grid=(NB,),
        in_specs=[
            pl.BlockSpec((NBLK, DN), lambda i: (i, 0)),
            pl.BlockSpec((EBLK, DE), lambda i: (i, 0)),
            pl.BlockSpec((DN, H * F), lambda i: (0, 0)),
            pl.BlockSpec((DE, H * F), lambda i: (0, 0)),
            pl.BlockSpec((1, H, 3 * F), lambda i: (0, 0, 0)),
        ],
        out_specs=[
            pl.BlockSpec((2, NBLK, W), lambda i: (0, i, 0)),
            pl.BlockSpec((NBLK, 16), lambda i: (i, 0)),
            pl.BlockSpec((EBLK, H), lambda i: (i, 0)),
        ],
        out_shape=[
            jax.ShapeDtypeStruct((2, N, W), jnp.float32),
            jax.ShapeDtypeStruct((N, 16), jnp.float32),
            jax.ShapeDtypeStruct((E, H), jnp.float32),
        ],
    )(node_feats, edge_feats, wn, we, attn)


C = 80
L = 16
NSUB = 16
EPT = E // NSUB
NCHUNK = EPT // C
NPT = 624
NTAIL = N - NPT * NSUB


def _sc_body(hc_hbm, ns_hbm, ae_hbm, src_hbm, dst_hbm, zw_hbm,
             s_out,
             src_v, dst_v, idx_v, ae_v, nsrc_v, ndst_v, exc0_v, exc1_v, row_v,
             s_sh, sem, sem2, sem3):
    c = lax.axis_index("c")
    sid = lax.axis_index("s")
    iota16 = lax.iota(jnp.int32, L)

    r0 = pl.multiple_of(sid * NPT, 8)
    pltpu.sync_copy(zw_hbm.at[pl.ds(r0, NPT)], s_sh.at[pl.ds(r0, NPT)])

    @pl.when(sid == 0)
    def _zero_tail():
        tail = NPT * NSUB
        pltpu.sync_copy(zw_hbm.at[pl.ds(tail, NTAIL)], s_sh.at[pl.ds(tail, NTAIL)])

    cnt_pat = jnp.where(iota16 == 2, 1.0, 0.0).astype(jnp.float32)
    mask0 = jnp.where(iota16 == 0, 1.0, 0.0).astype(jnp.float32)
    mask1 = jnp.where(iota16 == 1, 1.0, 0.0).astype(jnp.float32)
    plsc.subcore_barrier()

    def chunk(k, carry):
        base = pl.multiple_of(sid * EPT + k * C, 8)
        pltpu.sync_copy(src_hbm.at[pl.ds(base, C)], src_v)
        pltpu.sync_copy(dst_hbm.at[pl.ds(base, C)], dst_v)
        pltpu.sync_copy(ae_hbm.at[pl.ds(base * H, C * H)], ae_v)
        cp_ns = pltpu.async_copy(ns_hbm.at[src_v], nsrc_v, sem)
        cp_nd = pltpu.async_copy(ns_hbm.at[dst_v], ndst_v, sem2)

        def mkidx(g, carry2):
            sl = pl.ds(g * L, L)
            idx_v[sl] = src_v[sl] + c * N
            return carry2

        lax.fori_loop(0, C // L, mkidx, 0)
        cp_h = pltpu.async_copy(hc_hbm.at[idx_v], row_v, sem3)
        cp_ns.wait()
        cp_nd.wait()
        for g in range(C // L):
            sl = pl.ds(g * L, L)
            lane = jnp.full((L,), g * L, jnp.int32) + iota16
            for hh in range(2):
                col = jnp.full((L,), c * 2 + hh, jnp.int32)
                a_s = plsc.load_gather(nsrc_v, [lane, col])
                a_d = plsc.load_gather(ndst_v, [lane, col + 4])
                a_e = plsc.load_gather(ae_v, [lane * H + (c * 2 + hh)])
                a = a_s + a_d + a_e
                a = jnp.maximum(a, 0.2 * a)
                exv = jnp.exp(a)
                if hh == 0:
                    exc0_v[sl] = exv
                else:
                    exc1_v[sl] = exv
        cp_h.wait()

        def srow(j, carry2):
            jv = jnp.full((L,), j, jnp.int32)
            b0 = plsc.load_gather(exc0_v, [jv])
            b1 = plsc.load_gather(exc1_v, [jv])
            row_v[j, pl.ds(128, L)] = b0 * mask0 + b1 * mask1 + cnt_pat
            for kk in range(8):
                sl2 = pl.ds(kk * L, L)
                row_v[j, sl2] = row_v[j, sl2] * (b0 if kk < 4 else b1)
            return carry2

        lax.fori_loop(0, C, srow, 0)
        pltpu.sync_copy(row_v, s_sh.at[dst_v], add=True)
        return carry

    lax.fori_loop(0, NCHUNK, chunk, 0)
    plsc.subcore_barrier()
    pltpu.sync_copy(s_sh.at[pl.ds(r0, NPT)], s_out.at[c, pl.ds(r0, NPT)])

    @pl.when(sid == 0)
    def _copy_tail():
        tail = NPT * NSUB
        pltpu.sync_copy(s_sh.at[pl.ds(tail, NTAIL)], s_out.at[c, pl.ds(tail, NTAIL)])


def _sc_edge(hc2, ns, ae, src, dst):
    mesh = plsc.VectorSubcoreMesh(core_axis_name="c", subcore_axis_name="s")
    f = functools.partial(
        pl.kernel,
        out_type=jax.ShapeDtypeStruct((2, N, W), jnp.float32),
        mesh=mesh,
        compiler_params=pltpu.CompilerParams(needs_layout_passes=False, use_tc_tiling_on_sc=False),
        scratch_types=[
            pltpu.VMEM((C,), jnp.int32),
            pltpu.VMEM((C,), jnp.int32),
            pltpu.VMEM((C,), jnp.int32),
            pltpu.VMEM((C * H,), jnp.float32),
            pltpu.VMEM((C, 16), jnp.float32),
            pltpu.VMEM((C, 16), jnp.float32),
            pltpu.VMEM((C,), jnp.float32),
            pltpu.VMEM((C,), jnp.float32),
            pltpu.VMEM((C, W), jnp.float32),
            pltpu.VMEM_SHARED((N, W), jnp.float32),
            pltpu.SemaphoreType.DMA,
            pltpu.SemaphoreType.DMA,
            pltpu.SemaphoreType.DMA,
        ],
    )(_sc_body)
    zw = jnp.zeros((N, W), jnp.float32)
    return f(hc2, ns, ae.reshape(-1), src, dst, zw)


def _tc_final_body(s_r, out_r):
    cnt = s_r[0, :, 130]
    cfac = 1.0 / jnp.maximum(cnt, 1.0)
    outs = []
    for h in range(H):
        sc = h // 2
        hh = h % 2
        dh = s_r[sc, :, 128 + hh]
        w = cfac / jnp.maximum(dh, 1e-30)
        outs.append(s_r[sc, :, hh * F:(hh + 1) * F] * w[:, None])
    out_r[...] = jnp.concatenate(outs, axis=1)


def _tc_final(s_acc):
    return pl.pallas_call(
        _tc_final_body,
        grid=(NB,),
        in_specs=[
            pl.BlockSpec((2, NBLK, W), lambda i: (0, i, 0)),
        ],
        out_specs=pl.BlockSpec((NBLK, H * F), lambda i: (i, 0)),
        out_shape=jax.ShapeDtypeStruct((N, H * F), jnp.float32),
    )(s_acc)


def kernel(node_feats, edge_feats, edge_index, Wn, We, attn):
    hc, ns, ae = _tc_proj(node_feats, edge_feats, Wn, We, attn)
    src = edge_index[0]
    dst = edge_index[1]
    s_acc = _sc_edge(hc.reshape(2 * N, W), ns, ae, src, dst)
    return _tc_final(s_acc)


# speedup vs baseline: 35.4898x; 1.3083x over previous
import functools

import jax
import jax.numpy as jnp
from jax import lax
from jax.experimental import pallas as pl
from jax.experimental.pallas import tpu as pltpu
from jax.experimental.pallas import tpu_sc as plsc

N = 10000
E = 160000
H = 4
F = 64
DN = 256
DE = 16

NB = 10
NBLK = N // NB
EBLK = E // NB
W = 144


def _tc_proj_body(nf, ef, wn, we, attn, hc_r, ns_r, ae_r):
    hblk = jnp.dot(nf[...], wn[...], preferred_element_type=jnp.float32)
    ztail = jnp.zeros((NBLK, W - 128), jnp.float32)
    hc_r[0] = jnp.concatenate([hblk[:, :128], ztail], axis=1)
    hc_r[1] = jnp.concatenate([hblk[:, 128:], ztail], axis=1)
    att = attn[0]
    cols = []
    for h in range(H):
        hb = hblk[:, h * F:(h + 1) * F]
        cols.append(jnp.sum(hb * att[h, 0:F][None, :], axis=1, keepdims=True))
    for h in range(H):
        hb = hblk[:, h * F:(h + 1) * F]
        cols.append(jnp.sum(hb * att[h, F:2 * F][None, :], axis=1, keepdims=True))
    cols.append(jnp.zeros((NBLK, 8), jnp.float32))
    ns_r[...] = jnp.concatenate(cols, axis=1)
    wea_cols = [
        jnp.sum(we[...][:, h * F:(h + 1) * F] * att[h, 2 * F:3 * F][None, :],
                axis=1, keepdims=True)
        for h in range(H)
    ]
    we_a = jnp.concatenate(wea_cols, axis=1)
    ae_r[...] = jnp.dot(ef[...], we_a, preferred_element_type=jnp.float32)


def _tc_proj(node_feats, edge_feats, wn, we, attn):
    return pl.pallas_call(
        _tc_proj_body,
        grid=(NB,),
        in_specs=[
            pl.BlockSpec((NBLK, DN), lambda i: (i, 0)),
            pl.BlockSpec((EBLK, DE), lambda i: (i, 0)),
            pl.BlockSpec((DN, H * F), lambda i: (0, 0)),
            pl.BlockSpec((DE, H * F), lambda i: (0, 0)),
            pl.BlockSpec((1, H, 3 * F), lambda i: (0, 0, 0)),
        ],
        out_specs=[
            pl.BlockSpec((2, NBLK, W), lambda i: (0, i, 0)),
            pl.BlockSpec((NBLK, 16), lambda i: (i, 0)),
            pl.BlockSpec((EBLK, H), lambda i: (i, 0)),
        ],
        out_shape=[
            jax.ShapeDtypeStruct((2, N, W), jnp.float32),
            jax.ShapeDtypeStruct((N, 16), jnp.float32),
            jax.ShapeDtypeStruct((E, H), jnp.float32),
        ],
    )(node_feats, edge_feats, wn, we, attn)


C = 80
L = 16
NSUB = 16
EPT = E // NSUB
NCHUNK = EPT // C
NPT = 624
NTAIL = N - NPT * NSUB


def _sc_body(hc_hbm, ns_hbm, ae_hbm, src_hbm, dst_hbm, zw_hbm,
             s_out,
             src0_v, src1_v, dst0_v, dst1_v, sdst0_v, sdst1_v, idx_v,
             ae0_v, ae1_v, nsrc_v, ndst_v, exc0_v, exc1_v, row0_v, row1_v,
             s_sh, seml0, seml1, semg0, semg1, sems0, sems1):
    c = lax.axis_index("c")
    sid = lax.axis_index("s")
    iota16 = lax.iota(jnp.int32, L)
    src_v = (src0_v, src1_v)
    dst_v = (dst0_v, dst1_v)
    sdst_v = (sdst0_v, sdst1_v)
    ae_v = (ae0_v, ae1_v)
    row_v = (row0_v, row1_v)
    seml = (seml0, seml1)
    semg = (semg0, semg1)
    sems = (sems0, sems1)

    r0 = pl.multiple_of(sid * NPT, 8)
    pltpu.sync_copy(zw_hbm.at[pl.ds(r0, NPT)], s_sh.at[pl.ds(r0, NPT)])

    @pl.when(sid == 0)
    def _zero_tail():
        tail = NPT * NSUB
        pltpu.sync_copy(zw_hbm.at[pl.ds(tail, NTAIL)], s_sh.at[pl.ds(tail, NTAIL)])

    cnt_pat = jnp.where(iota16 == 2, 1.0, 0.0).astype(jnp.float32)
    mask0 = jnp.where(iota16 == 0, 1.0, 0.0).astype(jnp.float32)
    mask1 = jnp.where(iota16 == 1, 1.0, 0.0).astype(jnp.float32)
    e0 = pl.multiple_of(sid * EPT, 8)
    for b in range(2):
        pltpu.sync_copy(zw_hbm.at[pl.ds(0, C)], row_v[b])
        pltpu.sync_copy(src_hbm.at[pl.ds(e0, C)], sdst_v[b])
    plsc.subcore_barrier()
    for b in range(2):
        pltpu.async_copy(row_v[b], s_sh.at[sdst_v[b]], sems[b], add=True)
    pltpu.async_copy(src_hbm.at[pl.ds(e0, C)], src_v[0], seml[0])
    pltpu.async_copy(dst_hbm.at[pl.ds(e0, C)], dst_v[0], seml[0])
    pltpu.async_copy(ae_hbm.at[pl.ds(e0 * H, C * H)], ae_v[0], seml[0])

    def half(b, k, issue_next):
        base = pl.multiple_of(sid * EPT + k * C, 8)
        pltpu.make_async_copy(zw_hbm.at[pl.ds(0, C)], row_v[b], sems[b]).wait()
        pltpu.make_async_copy(src_hbm.at[pl.ds(base, C)], src_v[b], seml[b]).wait()
        pltpu.make_async_copy(dst_hbm.at[pl.ds(base, C)], dst_v[b], seml[b]).wait()
        pltpu.make_async_copy(ae_hbm.at[pl.ds(base * H, C * H)], ae_v[b], seml[b]).wait()

        def mkidx(g, carry2):
            sl = pl.ds(g * L, L)
            idx_v[sl] = src_v[b][sl] + c * N
            sdst_v[b][sl] = dst_v[b][sl]
            return carry2

        lax.fori_loop(0, C // L, mkidx, 0)
        g1 = pltpu.async_copy(ns_hbm.at[src_v[b]], nsrc_v, semg[b])
        g2 = pltpu.async_copy(ns_hbm.at[dst_v[b]], ndst_v, semg[b])
        g3 = pltpu.async_copy(hc_hbm.at[idx_v], row_v[b], semg[b])
        if issue_next:
            nbase = pl.multiple_of(sid * EPT + (k + 1) * C, 8)
            pltpu.async_copy(src_hbm.at[pl.ds(nbase, C)], src_v[1 - b], seml[1 - b])
            pltpu.async_copy(dst_hbm.at[pl.ds(nbase, C)], dst_v[1 - b], seml[1 - b])
            pltpu.async_copy(ae_hbm.at[pl.ds(nbase * H, C * H)], ae_v[1 - b], seml[1 - b])
        g1.wait()
        g2.wait()
        g3.wait()
        for g in range(C // L):
            sl = pl.ds(g * L, L)
            lane = jnp.full((L,), g * L, jnp.int32) + iota16
            for hh in range(2):
                col = jnp.full((L,), c * 2 + hh, jnp.int32)
                a_s = plsc.load_gather(nsrc_v, [lane, col])
                a_d = plsc.load_gather(ndst_v, [lane, col + 4])
                a_e = plsc.load_gather(ae_v[b], [lane * H + (c * 2 + hh)])
                a = a_s + a_d + a_e
                a = jnp.maximum(a, 0.2 * a)
                exv = jnp.exp(a)
                if hh == 0:
                    exc0_v[sl] = exv
                else:
                    exc1_v[sl] = exv

        def srow(j, carry2):
            jv = jnp.full((L,), j, jnp.int32)
            b0 = plsc.load_gather(exc0_v, [jv])
            b1 = plsc.load_gather(exc1_v, [jv])
            row_v[b][j, pl.ds(128, L)] = b0 * mask0 + b1 * mask1 + cnt_pat
            for kk in range(8):
                sl2 = pl.ds(kk * L, L)
                row_v[b][j, sl2] = row_v[b][j, sl2] * (b0 if kk < 4 else b1)
            return carry2

        lax.fori_loop(0, C, srow, 0)
        pltpu.async_copy(row_v[b], s_sh.at[sdst_v[b]], sems[b], add=True)

    def pair(m, carry):
        half(0, 2 * m, True)
        half(1, 2 * m + 1, True)
        return carry

    lax.fori_loop(0, (NCHUNK - 1) // 2, pair, 0)
    half(0, NCHUNK - 1, False)
    pltpu.make_async_copy(zw_hbm.at[pl.ds(0, C)], row_v[1], sems[1]).wait()
    pltpu.make_async_copy(zw_hbm.at[pl.ds(0, C)], row_v[0], sems[0]).wait()
    plsc.subcore_barrier()
    pltpu.sync_copy(s_sh.at[pl.ds(r0, NPT)], s_out.at[c, pl.ds(r0, NPT)])

    @pl.when(sid == 0)
    def _copy_tail():
        tail = NPT * NSUB
        pltpu.sync_copy(s_sh.at[pl.ds(tail, NTAIL)], s_out.at[c, pl.ds(tail, NTAIL)])


def _sc_edge(hc2, ns, ae, src, dst):
    mesh = plsc.VectorSubcoreMesh(core_axis_name="c", subcore_axis_name="s")
    f = functools.partial(
        pl.kernel,
        out_type=jax.ShapeDtypeStruct((2, N, W), jnp.float32),
        mesh=mesh,
        compiler_params=pltpu.CompilerParams(needs_layout_passes=False, use_tc_tiling_on_sc=False),
        scratch_types=[
            pltpu.VMEM((C,), jnp.int32),
            pltpu.VMEM((C,), jnp.int32),
            pltpu.VMEM((C,), jnp.int32),
            pltpu.VMEM((C,), jnp.int32),
            pltpu.VMEM((C,), jnp.int32),
            pltpu.VMEM((C,), jnp.int32),
            pltpu.VMEM((C,), jnp.int32),
            pltpu.VMEM((C * H,), jnp.float32),
            pltpu.VMEM((C * H,), jnp.float32),
            pltpu.VMEM((C, 16), jnp.float32),
            pltpu.VMEM((C, 16), jnp.float32),
            pltpu.VMEM((C,), jnp.float32),
            pltpu.VMEM((C,), jnp.float32),
            pltpu.VMEM((C, W), jnp.float32),
            pltpu.VMEM((C, W), jnp.float32),
            pltpu.VMEM_SHARED((N, W), jnp.float32),
            pltpu.SemaphoreType.DMA,
            pltpu.SemaphoreType.DMA,
            pltpu.SemaphoreType.DMA,
            pltpu.SemaphoreType.DMA,
            pltpu.SemaphoreType.DMA,
            pltpu.SemaphoreType.DMA,
        ],
    )(_sc_body)
    zw = jnp.zeros((N, W), jnp.float32)
    return f(hc2, ns, ae.reshape(-1), src, dst, zw)


def _tc_final_body(s_r, out_r):
    cnt = s_r[0, :, 130]
    cfac = 1.0 / jnp.maximum(cnt, 1.0)
    outs = []
    for h in range(H):
        sc = h // 2
        hh = h % 2
        dh = s_r[sc, :, 128 + hh]
        w = cfac / jnp.maximum(dh, 1e-30)
        outs.append(s_r[sc, :, hh * F:(hh + 1) * F] * w[:, None])
    out_r[...] = jnp.concatenate(outs, axis=1)


def _tc_final(s_acc):
    return pl.pallas_call(
        _tc_final_body,
        grid=(NB,),
        in_specs=[
            pl.BlockSpec((2, NBLK, W), lambda i: (0, i, 0)),
        ],
        out_specs=pl.BlockSpec((NBLK, H * F), lambda i: (i, 0)),
        out_shape=jax.ShapeDtypeStruct((N, H * F), jnp.float32),
    )(s_acc)


def kernel(node_feats, edge_feats, edge_index, Wn, We, attn):
    hc, ns, ae = _tc_proj(node_feats, edge_feats, Wn, We, attn)
    src = edge_index[0]
    dst = edge_index[1]
    s_acc = _sc_edge(hc.reshape(2 * N, W), ns, ae, src, dst)
    return _tc_final(s_acc)


# speedup vs baseline: 37.9952x; 1.0706x over previous
import functools

import jax
import jax.numpy as jnp
from jax import lax
from jax.experimental import pallas as pl
from jax.experimental.pallas import tpu as pltpu
from jax.experimental.pallas import tpu_sc as plsc

N = 10000
E = 160000
H = 4
F = 64
DN = 256
DE = 16

NB = 10
NBLK = N // NB
EBLK = E // NB
W = 144


def _tc_proj_body(nf, ef, wn, asd_m, we_a, hc_r, ns_r, ae_r):
    hblk = jnp.dot(nf[...], wn[...], preferred_element_type=jnp.float32)
    ztail = jnp.zeros((NBLK, W - 128), jnp.float32)
    hc_r[0] = jnp.concatenate([hblk[:, :128], ztail], axis=1)
    hc_r[1] = jnp.concatenate([hblk[:, 128:], ztail], axis=1)
    ns_r[...] = jnp.dot(hblk, asd_m[...], preferred_element_type=jnp.float32)
    ae_r[...] = jnp.dot(ef[...], we_a[...], preferred_element_type=jnp.float32)


def _tc_proj(node_feats, edge_feats, wn, we, attn):
    att = attn[0]
    eye4 = jnp.eye(H, dtype=jnp.float32)
    m_s = (att[:, :F, None] * eye4[:, None, :]).reshape(H * F, H)
    m_d = (att[:, F:2 * F, None] * eye4[:, None, :]).reshape(H * F, H)
    asd_m = jnp.concatenate([m_s, m_d, jnp.zeros((H * F, 8), jnp.float32)], axis=1)
    we_a = jnp.einsum("dhf,hf->dh", we.reshape(DE, H, F), att[:, 2 * F:])
    return pl.pallas_call(
        _tc_proj_body,
        grid=(NB,),
        in_specs=[
            pl.BlockSpec((NBLK, DN), lambda i: (i, 0)),
            pl.BlockSpec((EBLK, DE), lambda i: (i, 0)),
            pl.BlockSpec((DN, H * F), lambda i: (0, 0)),
            pl.BlockSpec((H * F, 16), lambda i: (0, 0)),
            pl.BlockSpec((DE, H), lambda i: (0, 0)),
        ],
        out_specs=[
            pl.BlockSpec((2, NBLK, W), lambda i: (0, i, 0)),
            pl.BlockSpec((NBLK, 16), lambda i: (i, 0)),
            pl.BlockSpec((EBLK, H), lambda i: (i, 0)),
        ],
        out_shape=[
            jax.ShapeDtypeStruct((2, N, W), jnp.float32),
            jax.ShapeDtypeStruct((N, 16), jnp.float32),
            jax.ShapeDtypeStruct((E, H), jnp.float32),
        ],
    )(node_feats, edge_feats, wn, asd_m, we_a)


C = 80
L = 16
NSUB = 16
EPT = E // NSUB
NCHUNK = EPT // C
NPT = 624
NTAIL = N - NPT * NSUB


def _sc_body(hc_hbm, ns_hbm, ae_hbm, src_hbm, dst_hbm, zw_hbm,
             s_out,
             src0_v, src1_v, dst0_v, dst1_v, sdst0_v, sdst1_v, idx0_v, idx1_v,
             ae0_v, ae1_v, nsrc0_v, nsrc1_v, ndst0_v, ndst1_v,
             exc0_v, exc1_v, row0_v, row1_v,
             s_sh, seml0, seml1, semg0, semg1, sems0, sems1):
    c = lax.axis_index("c")
    sid = lax.axis_index("s")
    iota16 = lax.iota(jnp.int32, L)
    src_v = (src0_v, src1_v)
    dst_v = (dst0_v, dst1_v)
    sdst_v = (sdst0_v, sdst1_v)
    idx_v = (idx0_v, idx1_v)
    ae_v = (ae0_v, ae1_v)
    nsrc_v = (nsrc0_v, nsrc1_v)
    ndst_v = (ndst0_v, ndst1_v)
    row_v = (row0_v, row1_v)
    seml = (seml0, seml1)
    semg = (semg0, semg1)
    sems = (sems0, sems1)

    r0 = pl.multiple_of(sid * NPT, 8)
    pltpu.sync_copy(zw_hbm.at[pl.ds(r0, NPT)], s_sh.at[pl.ds(r0, NPT)])

    @pl.when(sid == 0)
    def _zero_tail():
        tail = NPT * NSUB
        pltpu.sync_copy(zw_hbm.at[pl.ds(tail, NTAIL)], s_sh.at[pl.ds(tail, NTAIL)])

    cnt_pat = jnp.where(iota16 == 2, 1.0, 0.0).astype(jnp.float32)
    mask0 = jnp.where(iota16 == 0, 1.0, 0.0).astype(jnp.float32)
    mask1 = jnp.where(iota16 == 1, 1.0, 0.0).astype(jnp.float32)
    e0 = pl.multiple_of(sid * EPT, 8)
    pltpu.sync_copy(zw_hbm.at[pl.ds(0, C)], row_v[1])
    pltpu.sync_copy(src_hbm.at[pl.ds(e0, C)], sdst_v[1])
    plsc.subcore_barrier()
    pltpu.async_copy(row_v[1], s_sh.at[sdst_v[1]], sems[1], add=True)
    pltpu.sync_copy(src_hbm.at[pl.ds(e0, C)], src_v[0])
    pltpu.sync_copy(dst_hbm.at[pl.ds(e0, C)], dst_v[0])
    pltpu.sync_copy(ae_hbm.at[pl.ds(e0 * H, C * H)], ae_v[0])

    @plsc.parallel_loop(0, C // L, unroll=5)
    def mkidx0(g):
        sl = pl.ds(g * L, L)
        idx_v[0][sl] = src_v[0][sl] + c * N
        sdst_v[0][sl] = dst_v[0][sl]
    pltpu.async_copy(ns_hbm.at[src_v[0]], nsrc_v[0], semg[0])
    pltpu.async_copy(ns_hbm.at[dst_v[0]], ndst_v[0], semg[0])
    pltpu.async_copy(hc_hbm.at[idx_v[0]], row_v[0], semg[0])
    pltpu.async_copy(src_hbm.at[pl.ds(e0 + C, C)], src_v[1], seml[1])
    pltpu.async_copy(dst_hbm.at[pl.ds(e0 + C, C)], dst_v[1], seml[1])
    pltpu.async_copy(ae_hbm.at[pl.ds((e0 + C) * H, C * H)], ae_v[1], seml[1])

    def half(b, k, issue_next):
        pltpu.make_async_copy(ns_hbm.at[src_v[b]], nsrc_v[b], semg[b]).wait()
        pltpu.make_async_copy(ns_hbm.at[dst_v[b]], ndst_v[b], semg[b]).wait()
        pltpu.make_async_copy(hc_hbm.at[idx_v[b]], row_v[b], semg[b]).wait()
        for g in range(C // L):
            sl = pl.ds(g * L, L)
            lane = jnp.full((L,), g * L, jnp.int32) + iota16
            for hh in range(2):
                col = jnp.full((L,), c * 2 + hh, jnp.int32)
                a_s = plsc.load_gather(nsrc_v[b], [lane, col])
                a_d = plsc.load_gather(ndst_v[b], [lane, col + 4])
                a_e = plsc.load_gather(ae_v[b], [lane * H + (c * 2 + hh)])
                a = a_s + a_d + a_e
                a = jnp.maximum(a, 0.2 * a)
                exv = jnp.exp(a)
                if hh == 0:
                    exc0_v[sl] = exv
                else:
                    exc1_v[sl] = exv

        @plsc.parallel_loop(0, C, unroll=4)
        def srow(j):
            jv = jnp.full((L,), j, jnp.int32)
            b0 = plsc.load_gather(exc0_v, [jv])
            b1 = plsc.load_gather(exc1_v, [jv])
            row_v[b][j, pl.ds(128, L)] = b0 * mask0 + b1 * mask1 + cnt_pat
            for kk in range(8):
                sl2 = pl.ds(kk * L, L)
                row_v[b][j, sl2] = row_v[b][j, sl2] * (b0 if kk < 4 else b1)
        pltpu.async_copy(row_v[b], s_sh.at[sdst_v[b]], sems[b], add=True)
        if issue_next:
            nbase = pl.multiple_of(sid * EPT + (k + 1) * C, 8)
            pltpu.make_async_copy(src_hbm.at[pl.ds(nbase, C)], src_v[1 - b], seml[1 - b]).wait()
            pltpu.make_async_copy(dst_hbm.at[pl.ds(nbase, C)], dst_v[1 - b], seml[1 - b]).wait()
            pltpu.make_async_copy(ae_hbm.at[pl.ds(nbase * H, C * H)], ae_v[1 - b], seml[1 - b]).wait()
            pltpu.make_async_copy(zw_hbm.at[pl.ds(0, C)], row_v[1 - b], sems[1 - b]).wait()

            @plsc.parallel_loop(0, C // L, unroll=5)
            def mkidx(g):
                sl = pl.ds(g * L, L)
                idx_v[1 - b][sl] = src_v[1 - b][sl] + c * N
                sdst_v[1 - b][sl] = dst_v[1 - b][sl]
            pltpu.async_copy(ns_hbm.at[src_v[1 - b]], nsrc_v[1 - b], semg[1 - b])
            pltpu.async_copy(ns_hbm.at[dst_v[1 - b]], ndst_v[1 - b], semg[1 - b])
            pltpu.async_copy(hc_hbm.at[idx_v[1 - b]], row_v[1 - b], semg[1 - b])

            @pl.when(k + 2 < NCHUNK)
            def _next_linear():
                nnbase = pl.multiple_of(sid * EPT + (k + 2) * C, 8)
                pltpu.async_copy(src_hbm.at[pl.ds(nnbase, C)], src_v[b], seml[b])
                pltpu.async_copy(dst_hbm.at[pl.ds(nnbase, C)], dst_v[b], seml[b])
                pltpu.async_copy(ae_hbm.at[pl.ds(nnbase * H, C * H)], ae_v[b], seml[b])

    def pair(m, carry):
        half(0, 2 * m, True)
        half(1, 2 * m + 1, True)
        return carry

    lax.fori_loop(0, (NCHUNK - 1) // 2, pair, 0)
    half(0, NCHUNK - 1, False)
    pltpu.make_async_copy(zw_hbm.at[pl.ds(0, C)], row_v[1], sems[1]).wait()
    pltpu.make_async_copy(zw_hbm.at[pl.ds(0, C)], row_v[0], sems[0]).wait()
    plsc.subcore_barrier()
    pltpu.sync_copy(s_sh.at[pl.ds(r0, NPT)], s_out.at[c, pl.ds(r0, NPT)])

    @pl.when(sid == 0)
    def _copy_tail():
        tail = NPT * NSUB
        pltpu.sync_copy(s_sh.at[pl.ds(tail, NTAIL)], s_out.at[c, pl.ds(tail, NTAIL)])


def _sc_edge(hc2, ns, ae, src, dst):
    mesh = plsc.VectorSubcoreMesh(core_axis_name="c", subcore_axis_name="s")
    f = functools.partial(
        pl.kernel,
        out_type=jax.ShapeDtypeStruct((2, N, W), jnp.float32),
        mesh=mesh,
        compiler_params=pltpu.CompilerParams(needs_layout_passes=False, use_tc_tiling_on_sc=False),
        scratch_types=[
            pltpu.VMEM((C,), jnp.int32),
            pltpu.VMEM((C,), jnp.int32),
            pltpu.VMEM((C,), jnp.int32),
            pltpu.VMEM((C,), jnp.int32),
            pltpu.VMEM((C,), jnp.int32),
            pltpu.VMEM((C,), jnp.int32),
            pltpu.VMEM((C,), jnp.int32),
            pltpu.VMEM((C,), jnp.int32),
            pltpu.VMEM((C * H,), jnp.float32),
            pltpu.VMEM((C * H,), jnp.float32),
            pltpu.VMEM((C, 16), jnp.float32),
            pltpu.VMEM((C, 16), jnp.float32),
            pltpu.VMEM((C, 16), jnp.float32),
            pltpu.VMEM((C, 16), jnp.float32),
            pltpu.VMEM((C,), jnp.float32),
            pltpu.VMEM((C,), jnp.float32),
            pltpu.VMEM((C, W), jnp.float32),
            pltpu.VMEM((C, W), jnp.float32),
            pltpu.VMEM_SHARED((N, W), jnp.float32),
            pltpu.SemaphoreType.DMA,
            pltpu.SemaphoreType.DMA,
            pltpu.SemaphoreType.DMA,
            pltpu.SemaphoreType.DMA,
            pltpu.SemaphoreType.DMA,
            pltpu.SemaphoreType.DMA,
        ],
    )(_sc_body)
    zw = jnp.zeros((N, W), jnp.float32)
    return f(hc2, ns, ae.reshape(-1), src, dst, zw)


def _tc_final_body(s_r, out_r):
    cnt = s_r[0, :, 130]
    cfac = 1.0 / jnp.maximum(cnt, 1.0)
    outs = []
    for h in range(H):
        sc = h // 2
        hh = h % 2
        dh = s_r[sc, :, 128 + hh]
        w = cfac / jnp.maximum(dh, 1e-30)
        outs.append(s_r[sc, :, hh * F:(hh + 1) * F] * w[:, None])
    out_r[...] = jnp.concatenate(outs, axis=1)


def _tc_final(s_acc):
    return pl.pallas_call(
        _tc_final_body,
        grid=(NB,),
        in_specs=[
            pl.BlockSpec((2, NBLK, W), lambda i: (0, i, 0)),
        ],
        out_specs=pl.BlockSpec((NBLK, H * F), lambda i: (i, 0)),
        out_shape=jax.ShapeDtypeStruct((N, H * F), jnp.float32),
    )(s_acc)


def kernel(node_feats, edge_feats, edge_index, Wn, We, attn):
    hc, ns, ae = _tc_proj(node_feats, edge_feats, Wn, We, attn)
    src = edge_index[0]
    dst = edge_index[1]
    s_acc = _sc_edge(hc.reshape(2 * N, W), ns, ae, src, dst)
    return _tc_final(s_acc)


# speedup vs baseline: 39.7937x; 1.0473x over previous
import functools

import jax
import jax.numpy as jnp
from jax import lax
from jax.experimental import pallas as pl
from jax.experimental.pallas import tpu as pltpu
from jax.experimental.pallas import tpu_sc as plsc

N = 10000
E = 160000
H = 4
F = 64
DN = 256
DE = 16

NB = 10
NBLK = N // NB
EBLK = E // NB
W = 144


def _tc_proj_body(nf, ef, wn, asd_m, we_a, hc_r, ns_r, ae_r):
    hblk = jnp.dot(nf[...], wn[...], preferred_element_type=jnp.float32)
    ztail = jnp.zeros((NBLK, W - 128), jnp.float32)
    hc_r[0] = jnp.concatenate([hblk[:, :128], ztail], axis=1)
    hc_r[1] = jnp.concatenate([hblk[:, 128:], ztail], axis=1)
    ns_r[...] = jnp.dot(hblk, asd_m[...], preferred_element_type=jnp.float32)
    ae_r[...] = jnp.dot(ef[...], we_a[...], preferred_element_type=jnp.float32)


def _tc_proj(node_feats, edge_feats, wn, we, attn):
    att = attn[0]
    eye4 = jnp.eye(H, dtype=jnp.float32)
    m_s = (att[:, :F, None] * eye4[:, None, :]).reshape(H * F, H)
    m_d = (att[:, F:2 * F, None] * eye4[:, None, :]).reshape(H * F, H)
    asd_m = jnp.concatenate([m_s, m_d, jnp.zeros((H * F, 8), jnp.float32)], axis=1)
    we_a = jnp.einsum("dhf,hf->dh", we.reshape(DE, H, F), att[:, 2 * F:])
    return pl.pallas_call(
        _tc_proj_body,
        grid=(NB,),
        in_specs=[
            pl.BlockSpec((NBLK, DN), lambda i: (i, 0)),
            pl.BlockSpec((EBLK, DE), lambda i: (i, 0)),
            pl.BlockSpec((DN, H * F), lambda i: (0, 0)),
            pl.BlockSpec((H * F, 16), lambda i: (0, 0)),
            pl.BlockSpec((DE, H), lambda i: (0, 0)),
        ],
        out_specs=[
            pl.BlockSpec((2, NBLK, W), lambda i: (0, i, 0)),
            pl.BlockSpec((NBLK, 16), lambda i: (i, 0)),
            pl.BlockSpec((EBLK, H), lambda i: (i, 0)),
        ],
        out_shape=[
            jax.ShapeDtypeStruct((2, N, W), jnp.float32),
            jax.ShapeDtypeStruct((N, 16), jnp.float32),
            jax.ShapeDtypeStruct((E, H), jnp.float32),
        ],
    )(node_feats, edge_feats, wn, asd_m, we_a)


C = 80
L = 16
NSUB = 16
EPT = E // NSUB
NCHUNK = EPT // C
NPT = 624
NTAIL = N - NPT * NSUB


def _sc_body(hc_hbm, ns_hbm, ae_hbm, src_hbm, dst_hbm, zw_hbm,
             s_out,
             src0_v, src1_v, dst0_v, dst1_v, sdst0_v, sdst1_v, idx0_v, idx1_v,
             ae0_v, ae1_v, nsrc0_v, nsrc1_v, ndst0_v, ndst1_v,
             exc0_v, exc1_v, row0_v, row1_v,
             s_sh, seml0, seml1, semg0, semg1, sems0, sems1):
    c = lax.axis_index("c")
    sid = lax.axis_index("s")
    iota16 = lax.iota(jnp.int32, L)
    src_v = (src0_v, src1_v)
    dst_v = (dst0_v, dst1_v)
    sdst_v = (sdst0_v, sdst1_v)
    idx_v = (idx0_v, idx1_v)
    ae_v = (ae0_v, ae1_v)
    nsrc_v = (nsrc0_v, nsrc1_v)
    ndst_v = (ndst0_v, ndst1_v)
    row_v = (row0_v, row1_v)
    seml = (seml0, seml1)
    semg = (semg0, semg1)
    sems = (sems0, sems1)

    r0 = pl.multiple_of(sid * NPT, 8)
    pltpu.sync_copy(zw_hbm.at[pl.ds(r0, NPT)], s_sh.at[pl.ds(r0, NPT)])

    @pl.when(sid == 0)
    def _zero_tail():
        tail = NPT * NSUB
        pltpu.sync_copy(zw_hbm.at[pl.ds(tail, NTAIL)], s_sh.at[pl.ds(tail, NTAIL)])

    cnt_pat = jnp.where(iota16 == 2, 1.0, 0.0).astype(jnp.float32)
    mask0 = jnp.where(iota16 == 0, 1.0, 0.0).astype(jnp.float32)
    mask1 = jnp.where(iota16 == 1, 1.0, 0.0).astype(jnp.float32)
    e0 = pl.multiple_of(sid * EPT, 8)
    pltpu.sync_copy(zw_hbm.at[pl.ds(0, C)], row_v[1])
    pltpu.sync_copy(src_hbm.at[pl.ds(e0, C)], sdst_v[1])
    plsc.subcore_barrier()
    pltpu.async_copy(row_v[1], s_sh.at[sdst_v[1]], sems[1], add=True)
    pltpu.sync_copy(src_hbm.at[pl.ds(e0, C)], src_v[0])
    pltpu.sync_copy(dst_hbm.at[pl.ds(e0, C)], dst_v[0])
    pltpu.sync_copy(ae_hbm.at[pl.ds(e0 * H, C * H)], ae_v[0])

    @plsc.parallel_loop(0, C // L, unroll=5)
    def mkidx0(g):
        sl = pl.ds(g * L, L)
        idx_v[0][sl] = src_v[0][sl] + c * N
        sdst_v[0][sl] = dst_v[0][sl]
    pltpu.async_copy(ns_hbm.at[src_v[0]], nsrc_v[0], semg[0])
    pltpu.async_copy(ns_hbm.at[dst_v[0]], ndst_v[0], semg[0])
    pltpu.async_copy(hc_hbm.at[idx_v[0]], row_v[0], semg[0])
    pltpu.async_copy(src_hbm.at[pl.ds(e0 + C, C)], src_v[1], seml[1])
    pltpu.async_copy(dst_hbm.at[pl.ds(e0 + C, C)], dst_v[1], seml[1])
    pltpu.async_copy(ae_hbm.at[pl.ds((e0 + C) * H, C * H)], ae_v[1], seml[1])

    def half(b, k, issue_next):
        pltpu.make_async_copy(ns_hbm.at[src_v[b]], nsrc_v[b], semg[b]).wait()
        pltpu.make_async_copy(ns_hbm.at[dst_v[b]], ndst_v[b], semg[b]).wait()
        for g in range(C // L):
            sl = pl.ds(g * L, L)
            lane = jnp.full((L,), g * L, jnp.int32) + iota16
            for hh in range(2):
                col = jnp.full((L,), c * 2 + hh, jnp.int32)
                a_s = plsc.load_gather(nsrc_v[b], [lane, col])
                a_d = plsc.load_gather(ndst_v[b], [lane, col + 4])
                a_e = plsc.load_gather(ae_v[b], [lane * H + (c * 2 + hh)])
                a = a_s + a_d + a_e
                a = jnp.maximum(a, 0.2 * a)
                exv = jnp.exp(a)
                if hh == 0:
                    exc0_v[sl] = exv
                else:
                    exc1_v[sl] = exv

        pltpu.make_async_copy(hc_hbm.at[idx_v[b]], row_v[b], semg[b]).wait()

        @plsc.parallel_loop(0, C, unroll=4)
        def srow(j):
            jv = jnp.full((L,), j, jnp.int32)
            b0 = plsc.load_gather(exc0_v, [jv])
            b1 = plsc.load_gather(exc1_v, [jv])
            row_v[b][j, pl.ds(128, L)] = b0 * mask0 + b1 * mask1 + cnt_pat
            for kk in range(8):
                sl2 = pl.ds(kk * L, L)
                row_v[b][j, sl2] = row_v[b][j, sl2] * (b0 if kk < 4 else b1)
        pltpu.async_copy(row_v[b], s_sh.at[sdst_v[b]], sems[b], add=True)
        if issue_next:
            nbase = pl.multiple_of(sid * EPT + (k + 1) * C, 8)
            pltpu.make_async_copy(src_hbm.at[pl.ds(nbase, C)], src_v[1 - b], seml[1 - b]).wait()
            pltpu.make_async_copy(dst_hbm.at[pl.ds(nbase, C)], dst_v[1 - b], seml[1 - b]).wait()
            pltpu.make_async_copy(ae_hbm.at[pl.ds(nbase * H, C * H)], ae_v[1 - b], seml[1 - b]).wait()
            pltpu.make_async_copy(zw_hbm.at[pl.ds(0, C)], row_v[1 - b], sems[1 - b]).wait()

            @plsc.parallel_loop(0, C // L, unroll=5)
            def mkidx(g):
                sl = pl.ds(g * L, L)
                idx_v[1 - b][sl] = src_v[1 - b][sl] + c * N
                sdst_v[1 - b][sl] = dst_v[1 - b][sl]
            pltpu.async_copy(ns_hbm.at[src_v[1 - b]], nsrc_v[1 - b], semg[1 - b])
            pltpu.async_copy(ns_hbm.at[dst_v[1 - b]], ndst_v[1 - b], semg[1 - b])
            pltpu.async_copy(hc_hbm.at[idx_v[1 - b]], row_v[1 - b], semg[1 - b])

            @pl.when(k + 2 < NCHUNK)
            def _next_linear():
                nnbase = pl.multiple_of(sid * EPT + (k + 2) * C, 8)
                pltpu.async_copy(src_hbm.at[pl.ds(nnbase, C)], src_v[b], seml[b])
                pltpu.async_copy(dst_hbm.at[pl.ds(nnbase, C)], dst_v[b], seml[b])
                pltpu.async_copy(ae_hbm.at[pl.ds(nnbase * H, C * H)], ae_v[b], seml[b])

    def pair(m, carry):
        half(0, 2 * m, True)
        half(1, 2 * m + 1, True)
        return carry

    lax.fori_loop(0, (NCHUNK - 1) // 2, pair, 0)
    half(0, NCHUNK - 1, False)
    pltpu.make_async_copy(zw_hbm.at[pl.ds(0, C)], row_v[1], sems[1]).wait()
    pltpu.make_async_copy(zw_hbm.at[pl.ds(0, C)], row_v[0], sems[0]).wait()
    plsc.subcore_barrier()
    pltpu.sync_copy(s_sh.at[pl.ds(r0, NPT)], s_out.at[c, pl.ds(r0, NPT)])

    @pl.when(sid == 0)
    def _copy_tail():
        tail = NPT * NSUB
        pltpu.sync_copy(s_sh.at[pl.ds(tail, NTAIL)], s_out.at[c, pl.ds(tail, NTAIL)])


def _sc_edge(hc2, ns, ae, src, dst):
    mesh = plsc.VectorSubcoreMesh(core_axis_name="c", subcore_axis_name="s")
    f = functools.partial(
        pl.kernel,
        out_type=jax.ShapeDtypeStruct((2, N, W), jnp.float32),
        mesh=mesh,
        compiler_params=pltpu.CompilerParams(needs_layout_passes=False, use_tc_tiling_on_sc=False),
        scratch_types=[
            pltpu.VMEM((C,), jnp.int32),
            pltpu.VMEM((C,), jnp.int32),
            pltpu.VMEM((C,), jnp.int32),
            pltpu.VMEM((C,), jnp.int32),
            pltpu.VMEM((C,), jnp.int32),
            pltpu.VMEM((C,), jnp.int32),
            pltpu.VMEM((C,), jnp.int32),
            pltpu.VMEM((C,), jnp.int32),
            pltpu.VMEM((C * H,), jnp.float32),
            pltpu.VMEM((C * H,), jnp.float32),
            pltpu.VMEM((C, 16), jnp.float32),
            pltpu.VMEM((C, 16), jnp.float32),
            pltpu.VMEM((C, 16), jnp.float32),
            pltpu.VMEM((C, 16), jnp.float32),
            pltpu.VMEM((C,), jnp.float32),
            pltpu.VMEM((C,), jnp.float32),
            pltpu.VMEM((C, W), jnp.float32),
            pltpu.VMEM((C, W), jnp.float32),
            pltpu.VMEM_SHARED((N, W), jnp.float32),
            pltpu.SemaphoreType.DMA,
            pltpu.SemaphoreType.DMA,
            pltpu.SemaphoreType.DMA,
            pltpu.SemaphoreType.DMA,
            pltpu.SemaphoreType.DMA,
            pltpu.SemaphoreType.DMA,
        ],
    )(_sc_body)
    zw = jnp.zeros((N, W), jnp.float32)
    return f(hc2, ns, ae.reshape(-1), src, dst, zw)


def _tc_final_body(s_r, out_r):
    cnt = s_r[0, :, 130]
    cfac = 1.0 / jnp.maximum(cnt, 1.0)
    outs = []
    for h in range(H):
        sc = h // 2
        hh = h % 2
        dh = s_r[sc, :, 128 + hh]
        w = cfac / jnp.maximum(dh, 1e-30)
        outs.append(s_r[sc, :, hh * F:(hh + 1) * F] * w[:, None])
    out_r[...] = jnp.concatenate(outs, axis=1)


def _tc_final(s_acc):
    return pl.pallas_call(
        _tc_final_body,
        grid=(NB,),
        in_specs=[
            pl.BlockSpec((2, NBLK, W), lambda i: (0, i, 0)),
        ],
        out_specs=pl.BlockSpec((NBLK, H * F), lambda i: (i, 0)),
        out_shape=jax.ShapeDtypeStruct((N, H * F), jnp.float32),
    )(s_acc)


def kernel(node_feats, edge_feats, edge_index, Wn, We, attn):
    hc, ns, ae = _tc_proj(node_feats, edge_feats, Wn, We, attn)
    src = edge_index[0]
    dst = edge_index[1]
    s_acc = _sc_edge(hc.reshape(2 * N, W), ns, ae, src, dst)
    return _tc_final(s_acc)


# speedup vs baseline: 40.9276x; 1.0285x over previous
import functools

import jax
import jax.numpy as jnp
from jax import lax
from jax.experimental import pallas as pl
from jax.experimental.pallas import tpu as pltpu
from jax.experimental.pallas import tpu_sc as plsc

N = 10000
E = 160000
H = 4
F = 64
DN = 256
DE = 16

NB = 10
NBLK = N // NB
EBLK = E // NB
W = 144


def _tc_proj_body(nf, ef, wn, asd_m, we_a, hc_r, ns_r, ae_r):
    hblk = jnp.dot(nf[...], wn[...], preferred_element_type=jnp.float32)
    ztail = jnp.zeros((NBLK, W - 128), jnp.float32)
    hc_r[0] = jnp.concatenate([hblk[:, :128], ztail], axis=1)
    hc_r[1] = jnp.concatenate([hblk[:, 128:], ztail], axis=1)
    ns_r[...] = jnp.dot(hblk, asd_m[...], preferred_element_type=jnp.float32)
    ae_r[...] = jnp.dot(ef[...], we_a[...], preferred_element_type=jnp.float32)


def _tc_proj(node_feats, edge_feats, wn, we, attn):
    att = attn[0]
    eye4 = jnp.eye(H, dtype=jnp.float32)
    m_s = (att[:, :F, None] * eye4[:, None, :]).reshape(H * F, H)
    m_d = (att[:, F:2 * F, None] * eye4[:, None, :]).reshape(H * F, H)
    asd_m = jnp.concatenate([m_s, m_d, jnp.zeros((H * F, 8), jnp.float32)], axis=1)
    we_a = jnp.einsum("dhf,hf->dh", we.reshape(DE, H, F), att[:, 2 * F:])
    return pl.pallas_call(
        _tc_proj_body,
        grid=(NB,),
        in_specs=[
            pl.BlockSpec((NBLK, DN), lambda i: (i, 0)),
            pl.BlockSpec((EBLK, DE), lambda i: (i, 0)),
            pl.BlockSpec((DN, H * F), lambda i: (0, 0)),
            pl.BlockSpec((H * F, 16), lambda i: (0, 0)),
            pl.BlockSpec((DE, H), lambda i: (0, 0)),
        ],
        out_specs=[
            pl.BlockSpec((2, NBLK, W), lambda i: (0, i, 0)),
            pl.BlockSpec((NBLK, 16), lambda i: (i, 0)),
            pl.BlockSpec((EBLK, H), lambda i: (i, 0)),
        ],
        out_shape=[
            jax.ShapeDtypeStruct((2, N, W), jnp.float32),
            jax.ShapeDtypeStruct((N, 16), jnp.float32),
            jax.ShapeDtypeStruct((E, H), jnp.float32),
        ],
    )(node_feats, edge_feats, wn, asd_m, we_a)


C = 80
L = 16
NSUB = 16
EPT = E // NSUB
NCHUNK = EPT // C
NPT = 624
NTAIL = N - NPT * NSUB


def _sc_body(hc_hbm, ns_hbm, ae_hbm, src_hbm, dst_hbm, zw_hbm,
             s_out,
             src0_v, src1_v, dst0_v, dst1_v, sdst0_v, sdst1_v, idx0_v, idx1_v,
             ae0_v, ae1_v, nsrc0_v, nsrc1_v, ndst0_v, ndst1_v,
             exc0_v, exc1_v, row0_v, row1_v,
             s_sh, seml0, seml1, semg0, semg1, semh0, semh1, sems0, sems1):
    c = lax.axis_index("c")
    sid = lax.axis_index("s")
    iota16 = lax.iota(jnp.int32, L)
    src_v = (src0_v, src1_v)
    dst_v = (dst0_v, dst1_v)
    sdst_v = (sdst0_v, sdst1_v)
    idx_v = (idx0_v, idx1_v)
    ae_v = (ae0_v, ae1_v)
    nsrc_v = (nsrc0_v, nsrc1_v)
    ndst_v = (ndst0_v, ndst1_v)
    row_v = (row0_v, row1_v)
    seml = (seml0, seml1)
    semg = (semg0, semg1)
    semh = (semh0, semh1)
    sems = (sems0, sems1)

    r0 = pl.multiple_of(sid * NPT, 8)
    pltpu.sync_copy(zw_hbm.at[pl.ds(r0, NPT)], s_sh.at[pl.ds(r0, NPT)])

    @pl.when(sid == 0)
    def _zero_tail():
        tail = NPT * NSUB
        pltpu.sync_copy(zw_hbm.at[pl.ds(tail, NTAIL)], s_sh.at[pl.ds(tail, NTAIL)])

    cnt_pat = jnp.where(iota16 == 2, 1.0, 0.0).astype(jnp.float32)
    mask0 = jnp.where(iota16 == 0, 1.0, 0.0).astype(jnp.float32)
    mask1 = jnp.where(iota16 == 1, 1.0, 0.0).astype(jnp.float32)
    e0 = pl.multiple_of(sid * EPT, 8)
    pltpu.sync_copy(zw_hbm.at[pl.ds(0, C)], row_v[1])
    pltpu.sync_copy(src_hbm.at[pl.ds(e0, C)], sdst_v[1])
    plsc.subcore_barrier()
    pltpu.async_copy(row_v[1], s_sh.at[sdst_v[1]], sems[1], add=True)
    pltpu.sync_copy(src_hbm.at[pl.ds(e0, C)], src_v[0])
    pltpu.sync_copy(dst_hbm.at[pl.ds(e0, C)], dst_v[0])
    pltpu.sync_copy(ae_hbm.at[pl.ds(e0 * H, C * H)], ae_v[0])

    @plsc.parallel_loop(0, C // L, unroll=5)
    def mkidx0(g):
        sl = pl.ds(g * L, L)
        idx_v[0][sl] = src_v[0][sl] + c * N
        sdst_v[0][sl] = dst_v[0][sl]
    pltpu.async_copy(ns_hbm.at[src_v[0]], nsrc_v[0], semg[0])
    pltpu.async_copy(ns_hbm.at[dst_v[0]], ndst_v[0], semg[0])
    pltpu.async_copy(hc_hbm.at[idx_v[0]], row_v[0], semh[0])
    pltpu.async_copy(src_hbm.at[pl.ds(e0 + C, C)], src_v[1], seml[1])
    pltpu.async_copy(dst_hbm.at[pl.ds(e0 + C, C)], dst_v[1], seml[1])
    pltpu.async_copy(ae_hbm.at[pl.ds((e0 + C) * H, C * H)], ae_v[1], seml[1])

    def half(b, k, issue_next):
        pltpu.make_async_copy(ns_hbm.at[src_v[b]], nsrc_v[b], semg[b]).wait()
        pltpu.make_async_copy(ns_hbm.at[dst_v[b]], ndst_v[b], semg[b]).wait()
        for g in range(C // L):
            sl = pl.ds(g * L, L)
            lane = jnp.full((L,), g * L, jnp.int32) + iota16
            for hh in range(2):
                col = jnp.full((L,), c * 2 + hh, jnp.int32)
                a_s = plsc.load_gather(nsrc_v[b], [lane, col])
                a_d = plsc.load_gather(ndst_v[b], [lane, col + 4])
                a_e = plsc.load_gather(ae_v[b], [lane * H + (c * 2 + hh)])
                a = a_s + a_d + a_e
                a = jnp.maximum(a, 0.2 * a)
                exv = jnp.exp(a)
                if hh == 0:
                    exc0_v[sl] = exv
                else:
                    exc1_v[sl] = exv

        pltpu.make_async_copy(hc_hbm.at[idx_v[b]], row_v[b], semh[b]).wait()

        @plsc.parallel_loop(0, C, unroll=4)
        def srow(j):
            jv = jnp.full((L,), j, jnp.int32)
            b0 = plsc.load_gather(exc0_v, [jv])
            b1 = plsc.load_gather(exc1_v, [jv])
            row_v[b][j, pl.ds(128, L)] = b0 * mask0 + b1 * mask1 + cnt_pat
            for kk in range(8):
                sl2 = pl.ds(kk * L, L)
                row_v[b][j, sl2] = row_v[b][j, sl2] * (b0 if kk < 4 else b1)
        pltpu.async_copy(row_v[b], s_sh.at[sdst_v[b]], sems[b], add=True)
        if issue_next:
            nbase = pl.multiple_of(sid * EPT + (k + 1) * C, 8)
            pltpu.make_async_copy(src_hbm.at[pl.ds(nbase, C)], src_v[1 - b], seml[1 - b]).wait()
            pltpu.make_async_copy(dst_hbm.at[pl.ds(nbase, C)], dst_v[1 - b], seml[1 - b]).wait()
            pltpu.make_async_copy(ae_hbm.at[pl.ds(nbase * H, C * H)], ae_v[1 - b], seml[1 - b]).wait()
            pltpu.make_async_copy(zw_hbm.at[pl.ds(0, C)], row_v[1 - b], sems[1 - b]).wait()

            @plsc.parallel_loop(0, C // L, unroll=5)
            def mkidx(g):
                sl = pl.ds(g * L, L)
                idx_v[1 - b][sl] = src_v[1 - b][sl] + c * N
                sdst_v[1 - b][sl] = dst_v[1 - b][sl]
            pltpu.async_copy(ns_hbm.at[src_v[1 - b]], nsrc_v[1 - b], semg[1 - b])
            pltpu.async_copy(ns_hbm.at[dst_v[1 - b]], ndst_v[1 - b], semg[1 - b])
            pltpu.async_copy(hc_hbm.at[idx_v[1 - b]], row_v[1 - b], semh[1 - b])

            @pl.when(k + 2 < NCHUNK)
            def _next_linear():
                nnbase = pl.multiple_of(sid * EPT + (k + 2) * C, 8)
                pltpu.async_copy(src_hbm.at[pl.ds(nnbase, C)], src_v[b], seml[b])
                pltpu.async_copy(dst_hbm.at[pl.ds(nnbase, C)], dst_v[b], seml[b])
                pltpu.async_copy(ae_hbm.at[pl.ds(nnbase * H, C * H)], ae_v[b], seml[b])

    def pair(m, carry):
        half(0, 2 * m, True)
        half(1, 2 * m + 1, True)
        return carry

    lax.fori_loop(0, (NCHUNK - 1) // 2, pair, 0)
    half(0, NCHUNK - 1, False)
    pltpu.make_async_copy(zw_hbm.at[pl.ds(0, C)], row_v[1], sems[1]).wait()
    pltpu.make_async_copy(zw_hbm.at[pl.ds(0, C)], row_v[0], sems[0]).wait()
    plsc.subcore_barrier()
    pltpu.sync_copy(s_sh.at[pl.ds(r0, NPT)], s_out.at[c, pl.ds(r0, NPT)])

    @pl.when(sid == 0)
    def _copy_tail():
        tail = NPT * NSUB
        pltpu.sync_copy(s_sh.at[pl.ds(tail, NTAIL)], s_out.at[c, pl.ds(tail, NTAIL)])


def _sc_edge(hc2, ns, ae, src, dst):
    mesh = plsc.VectorSubcoreMesh(core_axis_name="c", subcore_axis_name="s")
    f = functools.partial(
        pl.kernel,
        out_type=jax.ShapeDtypeStruct((2, N, W), jnp.float32),
        mesh=mesh,
        compiler_params=pltpu.CompilerParams(needs_layout_passes=False, use_tc_tiling_on_sc=False),
        scratch_types=[
            pltpu.VMEM((C,), jnp.int32),
            pltpu.VMEM((C,), jnp.int32),
            pltpu.VMEM((C,), jnp.int32),
            pltpu.VMEM((C,), jnp.int32),
            pltpu.VMEM((C,), jnp.int32),
            pltpu.VMEM((C,), jnp.int32),
            pltpu.VMEM((C,), jnp.int32),
            pltpu.VMEM((C,), jnp.int32),
            pltpu.VMEM((C * H,), jnp.float32),
            pltpu.VMEM((C * H,), jnp.float32),
            pltpu.VMEM((C, 16), jnp.float32),
            pltpu.VMEM((C, 16), jnp.float32),
            pltpu.VMEM((C, 16), jnp.float32),
            pltpu.VMEM((C, 16), jnp.float32),
            pltpu.VMEM((C,), jnp.float32),
            pltpu.VMEM((C,), jnp.float32),
            pltpu.VMEM((C, W), jnp.float32),
            pltpu.VMEM((C, W), jnp.float32),
            pltpu.VMEM_SHARED((N, W), jnp.float32),
            pltpu.SemaphoreType.DMA,
            pltpu.SemaphoreType.DMA,
            pltpu.SemaphoreType.DMA,
            pltpu.SemaphoreType.DMA,
            pltpu.SemaphoreType.DMA,
            pltpu.SemaphoreType.DMA,
            pltpu.SemaphoreType.DMA,
            pltpu.SemaphoreType.DMA,
        ],
    )(_sc_body)
    zw = jnp.zeros((N, W), jnp.float32)
    return f(hc2, ns, ae.reshape(-1), src, dst, zw)


def _tc_final_body(s_r, out_r):
    cnt = s_r[0, :, 130]
    cfac = 1.0 / jnp.maximum(cnt, 1.0)
    outs = []
    for h in range(H):
        sc = h // 2
        hh = h % 2
        dh = s_r[sc, :, 128 + hh]
        w = cfac / jnp.maximum(dh, 1e-30)
        outs.append(s_r[sc, :, hh * F:(hh + 1) * F] * w[:, None])
    out_r[...] = jnp.concatenate(outs, axis=1)


def _tc_final(s_acc):
    return pl.pallas_call(
        _tc_final_body,
        grid=(NB,),
        in_specs=[
            pl.BlockSpec((2, NBLK, W), lambda i: (0, i, 0)),
        ],
        out_specs=pl.BlockSpec((NBLK, H * F), lambda i: (i, 0)),
        out_shape=jax.ShapeDtypeStruct((N, H * F), jnp.float32),
    )(s_acc)


def kernel(node_feats, edge_feats, edge_index, Wn, We, attn):
    hc, ns, ae = _tc_proj(node_feats, edge_feats, Wn, We, attn)
    src = edge_index[0]
    dst = edge_index[1]
    s_acc = _sc_edge(hc.reshape(2 * N, W), ns, ae, src, dst)
    return _tc_final(s_acc)
